# baseline (speedup 1.0000x reference)
import jax
import jax.numpy as jnp
from jax import lax
from jax.experimental import pallas as pl
from jax.experimental.pallas import tpu as pltpu

B, H, W = 8192, 64, 64
BLOCK_B = 256
NUM_BLOCKS = B // BLOCK_B


def _loss_block_kernel(x_ref, out_ref):
    xb = x_ref[...]
    m = jnp.max(xb, axis=(1, 2), keepdims=True)
    jx = lax.broadcasted_iota(jnp.int32, (1, H, W), 1)
    kx = lax.broadcasted_iota(jnp.int32, (1, H, W), 2)
    flat = jx * W + kx
    idx = jnp.min(jnp.where(xb == m, flat, H * W), axis=(1, 2),
                  keepdims=True)
    mx = (idx // W).astype(jnp.float32)
    my = (idx % W).astype(jnp.float32)
    r = jnp.sum(xb, axis=1)
    c = jnp.sum(xb, axis=2)
    kf = lax.broadcasted_iota(jnp.int32, (1, W), 1).astype(jnp.float32)
    jf = lax.broadcasted_iota(jnp.int32, (1, H), 1).astype(jnp.float32)
    s0 = jnp.sum(r, axis=1, keepdims=True)
    sk = jnp.sum(r * kf, axis=1, keepdims=True)
    sj = jnp.sum(c * jf, axis=1, keepdims=True)
    cs = jnp.sum(xb, axis=0)
    jf2 = (jx * jx + kx * kx).astype(jnp.float32)[0]
    s2g = jnp.sum(cs * jf2)
    mx2 = mx[:, 0, :]
    my2 = my[:, 0, :]
    loss_b = (mx2 * mx2 + my2 * my2) * s0 - 2.0 * (mx2 * sj + my2 * sk)
    out_ref[...] = jnp.full((1, 1, 128), jnp.sum(loss_b) + s2g,
                            dtype=jnp.float32)


def kernel(x):
    partials = pl.pallas_call(
        _loss_block_kernel,
        grid=(NUM_BLOCKS,),
        in_specs=[
            pl.BlockSpec((BLOCK_B, H, W), lambda i: (i, 0, 0)),
        ],
        out_specs=pl.BlockSpec((1, 1, 128), lambda i: (i, 0, 0)),
        out_shape=jax.ShapeDtypeStruct((NUM_BLOCKS, 1, 128), jnp.float32),
        compiler_params=pltpu.CompilerParams(
            dimension_semantics=("parallel",),
        ),
    )(x)
    return jnp.sum(partials[:, 0, 0]).reshape(1)


# speedup vs baseline: 4.3276x; 4.3276x over previous
import jax
import jax.numpy as jnp
from jax import lax
from jax.experimental import pallas as pl
from jax.experimental.pallas import tpu as pltpu

B, H, W = 8192, 64, 64
HW = H * W
BLOCK_LANES = 256
NUM_BLOCKS = B // BLOCK_LANES


def _loss_block_kernel(x_ref, out_ref):
    xb = x_ref[...]
    m = jnp.max(xb, axis=0, keepdims=True)
    r = lax.broadcasted_iota(jnp.int32, (HW, 1), 0)
    idx = jnp.min(jnp.where(xb == m, r, HW), axis=0, keepdims=True)
    mx = (idx // W).astype(jnp.float32)
    my = (idx % W).astype(jnp.float32)
    jf = (r // W).astype(jnp.float32)
    kf = (r % W).astype(jnp.float32)
    s0 = jnp.sum(xb, axis=0, keepdims=True)
    sj = jnp.sum(xb * jf, axis=0, keepdims=True)
    sk = jnp.sum(xb * kf, axis=0, keepdims=True)
    s2 = jnp.sum(xb * (jf * jf + kf * kf), axis=0, keepdims=True)
    loss_l = (mx * mx + my * my) * s0 - 2.0 * (mx * sj + my * sk) + s2
    out_ref[...] = jnp.full((1, 1, 128), jnp.sum(loss_l), dtype=jnp.float32)


def kernel(x):
    xt = x.transpose(1, 2, 0).reshape(HW, B)
    partials = pl.pallas_call(
        _loss_block_kernel,
        grid=(NUM_BLOCKS,),
        in_specs=[
            pl.BlockSpec((HW, BLOCK_LANES), lambda i: (0, i)),
        ],
        out_specs=pl.BlockSpec((1, 1, 128), lambda i: (i, 0, 0)),
        out_shape=jax.ShapeDtypeStruct((NUM_BLOCKS, 1, 128), jnp.float32),
        compiler_params=pltpu.CompilerParams(
            dimension_semantics=("parallel",),
        ),
    )(xt)
    return jnp.sum(partials[:, 0, 0]).reshape(1)


# speedup vs baseline: 6.7419x; 1.5579x over previous
import jax
import jax.numpy as jnp
from jax import lax
from jax.experimental import pallas as pl
from jax.experimental.pallas import tpu as pltpu

B, H, W = 8192, 64, 64
HW = H * W
BLOCK_LANES = 256
NUM_BLOCKS = B // BLOCK_LANES


def _loss_block_kernel(x_ref, out_ref):
    xb = x_ref[...]
    m = jnp.max(xb, axis=0, keepdims=True)
    r = lax.broadcasted_iota(jnp.int32, (HW, 1), 0)
    idx = jnp.min(jnp.where(xb == m, r, HW), axis=0, keepdims=True)
    mx = (idx // W).astype(jnp.float32)
    my = (idx % W).astype(jnp.float32)
    x3 = xb.reshape(H, W, -1)
    g = jnp.sum(x3, axis=1)
    gk = jnp.sum(x3, axis=0)
    jc = lax.broadcasted_iota(jnp.int32, (H, 1), 0).astype(jnp.float32)
    s0 = jnp.sum(g, axis=0, keepdims=True)
    sj = jnp.sum(g * jc, axis=0, keepdims=True)
    sj2 = jnp.sum(g * (jc * jc), axis=0, keepdims=True)
    sk = jnp.sum(gk * jc, axis=0, keepdims=True)
    sk2 = jnp.sum(gk * (jc * jc), axis=0, keepdims=True)
    loss_l = ((mx * mx + my * my) * s0 - 2.0 * (mx * sj + my * sk)
              + sj2 + sk2)
    out_ref[...] = jnp.full((1, 1, 128), jnp.sum(loss_l), dtype=jnp.float32)


def kernel(x):
    xt = x.transpose(1, 2, 0).reshape(HW, B)
    partials = pl.pallas_call(
        _loss_block_kernel,
        grid=(NUM_BLOCKS,),
        in_specs=[
            pl.BlockSpec((HW, BLOCK_LANES), lambda i: (0, i)),
        ],
        out_specs=pl.BlockSpec((1, 1, 128), lambda i: (i, 0, 0)),
        out_shape=jax.ShapeDtypeStruct((NUM_BLOCKS, 1, 128), jnp.float32),
        compiler_params=pltpu.CompilerParams(
            dimension_semantics=("parallel",),
        ),
    )(xt)
    return jnp.sum(partials[:, 0, 0]).reshape(1)


# speedup vs baseline: 6.9352x; 1.0287x over previous
import jax
import jax.numpy as jnp
from jax import lax
from jax.experimental import pallas as pl
from jax.experimental.pallas import tpu as pltpu

B, H, W = 8192, 64, 64
HW = H * W
BLOCK_LANES = 256
NUM_BLOCKS = B // BLOCK_LANES


def _loss_block_kernel(x_ref, out_ref):
    xb = x_ref[...]
    m = jnp.max(xb, axis=0, keepdims=True)
    r = lax.broadcasted_iota(jnp.int32, (HW, 1), 0)
    rf = r.astype(jnp.float32)
    idx = jnp.min(jnp.where(xb == m, rf, float(HW)), axis=0,
                  keepdims=True)
    mx = jnp.floor(idx * (1.0 / W))
    my = idx - W * mx
    x3 = xb.reshape(H, W, -1)
    g = jnp.sum(x3, axis=1)
    gk = jnp.sum(x3, axis=0)
    jc = lax.broadcasted_iota(jnp.int32, (H, 1), 0).astype(jnp.float32)
    s0 = jnp.sum(g, axis=0, keepdims=True)
    sj = jnp.sum(g * jc, axis=0, keepdims=True)
    sj2 = jnp.sum(g * (jc * jc), axis=0, keepdims=True)
    sk = jnp.sum(gk * jc, axis=0, keepdims=True)
    sk2 = jnp.sum(gk * (jc * jc), axis=0, keepdims=True)
    loss_l = ((mx * mx + my * my) * s0 - 2.0 * (mx * sj + my * sk)
              + sj2 + sk2)
    out_ref[...] = jnp.full((1, 1, 128), jnp.sum(loss_l), dtype=jnp.float32)


def kernel(x):
    xt = x.transpose(1, 2, 0).reshape(HW, B)
    partials = pl.pallas_call(
        _loss_block_kernel,
        grid=(NUM_BLOCKS,),
        in_specs=[
            pl.BlockSpec((HW, BLOCK_LANES), lambda i: (0, i)),
        ],
        out_specs=pl.BlockSpec((1, 1, 128), lambda i: (i, 0, 0)),
        out_shape=jax.ShapeDtypeStruct((NUM_BLOCKS, 1, 128), jnp.float32),
        compiler_params=pltpu.CompilerParams(
            dimension_semantics=("parallel",),
        ),
    )(xt)
    return jnp.sum(partials[:, 0, 0]).reshape(1)


# speedup vs baseline: 8.5224x; 1.2289x over previous
import jax
import jax.numpy as jnp
from jax import lax
from jax.experimental import pallas as pl
from jax.experimental.pallas import tpu as pltpu

B, H, W = 8192, 64, 64
HW = H * W
BLOCK_LANES = 512
NUM_BLOCKS = B // BLOCK_LANES


def _loss_block_kernel(x_ref, out_ref):
    xb = x_ref[...]
    m = jnp.max(xb, axis=0, keepdims=True)
    r = lax.broadcasted_iota(jnp.int32, (HW, 1), 0)
    rf = r.astype(jnp.float32)
    idx = jnp.min(jnp.where(xb == m, rf, float(HW)), axis=0,
                  keepdims=True)
    mx = jnp.floor(idx * (1.0 / W))
    my = idx - W * mx
    x3 = xb.reshape(H, W, -1)
    g = jnp.sum(x3, axis=1)
    gk = jnp.sum(x3, axis=0)
    jc = lax.broadcasted_iota(jnp.int32, (H, 1), 0).astype(jnp.float32)
    s0 = jnp.sum(g, axis=0, keepdims=True)
    sj = jnp.sum(g * jc, axis=0, keepdims=True)
    sj2 = jnp.sum(g * (jc * jc), axis=0, keepdims=True)
    sk = jnp.sum(gk * jc, axis=0, keepdims=True)
    sk2 = jnp.sum(gk * (jc * jc), axis=0, keepdims=True)
    loss_l = ((mx * mx + my * my) * s0 - 2.0 * (mx * sj + my * sk)
              + sj2 + sk2)
    out_ref[...] = jnp.full((1, 1, 128), jnp.sum(loss_l), dtype=jnp.float32)


def kernel(x):
    xt = x.transpose(1, 2, 0).reshape(HW, B)
    partials = pl.pallas_call(
        _loss_block_kernel,
        grid=(NUM_BLOCKS,),
        in_specs=[
            pl.BlockSpec((HW, BLOCK_LANES), lambda i: (0, i)),
        ],
        out_specs=pl.BlockSpec((1, 1, 128), lambda i: (i, 0, 0)),
        out_shape=jax.ShapeDtypeStruct((NUM_BLOCKS, 1, 128), jnp.float32),
        compiler_params=pltpu.CompilerParams(
            dimension_semantics=("parallel",),
        ),
    )(xt)
    return jnp.sum(partials[:, 0, 0]).reshape(1)


# speedup vs baseline: 9.2046x; 1.0801x over previous
import jax
import jax.numpy as jnp
from jax import lax
from jax.experimental import pallas as pl
from jax.experimental.pallas import tpu as pltpu

B, H, W = 8192, 64, 64
HW = H * W
BLOCK_LANES = 1024
NUM_BLOCKS = B // BLOCK_LANES


def _loss_block_kernel(x_ref, out_ref):
    xb = x_ref[...]
    m = jnp.max(xb, axis=0, keepdims=True)
    r = lax.broadcasted_iota(jnp.int32, (HW, 1), 0)
    rf = r.astype(jnp.float32)
    idx = jnp.min(jnp.where(xb == m, rf, float(HW)), axis=0,
                  keepdims=True)
    mx = jnp.floor(idx * (1.0 / W))
    my = idx - W * mx
    x3 = xb.reshape(H, W, -1)
    g = jnp.sum(x3, axis=1)
    gk = jnp.sum(x3, axis=0)
    jc = lax.broadcasted_iota(jnp.int32, (H, 1), 0).astype(jnp.float32)
    s0 = jnp.sum(g, axis=0, keepdims=True)
    sj = jnp.sum(g * jc, axis=0, keepdims=True)
    sj2 = jnp.sum(g * (jc * jc), axis=0, keepdims=True)
    sk = jnp.sum(gk * jc, axis=0, keepdims=True)
    sk2 = jnp.sum(gk * (jc * jc), axis=0, keepdims=True)
    loss_l = ((mx * mx + my * my) * s0 - 2.0 * (mx * sj + my * sk)
              + sj2 + sk2)
    out_ref[...] = jnp.full((1, 1, 128), jnp.sum(loss_l), dtype=jnp.float32)


def kernel(x):
    xt = x.transpose(1, 2, 0).reshape(HW, B)
    partials = pl.pallas_call(
        _loss_block_kernel,
        grid=(NUM_BLOCKS,),
        in_specs=[
            pl.BlockSpec((HW, BLOCK_LANES), lambda i: (0, i)),
        ],
        out_specs=pl.BlockSpec((1, 1, 128), lambda i: (i, 0, 0)),
        out_shape=jax.ShapeDtypeStruct((NUM_BLOCKS, 1, 128), jnp.float32),
        compiler_params=pltpu.CompilerParams(
            dimension_semantics=("parallel",),
        ),
    )(xt)
    return jnp.sum(partials[:, 0, 0]).reshape(1)
